# baseline (speedup 1.0000x reference)
import functools

import jax
import jax.numpy as jnp
from jax import lax
from jax.experimental import pallas as pl
from jax.experimental.pallas import tpu as pltpu
from jax.experimental.pallas import tpu_sc as plsc

NC = 2
NS = 16
NW = NC * NS

CK = 64
IDX_DMA = 128


def _sc_gather_pool(toks, c3, clast, emb, dec, *, V, D, T, BL, B):
    ppw = BL // NW
    nchunk = ppw // CK
    ckt = CK * T
    ndma = (ckt + IDX_DMA - 1) // IDX_DMA
    qpw = B // NW

    mesh = plsc.VectorSubcoreMesh(core_axis_name="c", subcore_axis_name="s",
                                  num_cores=NC, num_subcores=NS)

    @functools.partial(
        pl.kernel,
        out_type=[jax.ShapeDtypeStruct((4, BL, D), jnp.bfloat16),
                  jax.ShapeDtypeStruct((B, D), jnp.bfloat16)],
        mesh=mesh,
        compiler_params=pltpu.CompilerParams(use_tc_tiling_on_sc=False,
                                             needs_layout_passes=False),
        scratch_types=[
            pltpu.VMEM((2, ckt), jnp.int32),
            pltpu.VMEM((2, ckt, D), jnp.bfloat16),
            pltpu.VMEM((2, CK, D), jnp.bfloat16),
            pltpu.SemaphoreType.DMA,
            pltpu.SemaphoreType.DMA,
            pltpu.SemaphoreType.DMA,
            pltpu.SemaphoreType.DMA,
            pltpu.SemaphoreType.DMA,
            pltpu.SemaphoreType.DMA,
        ],
    )
    def sc_kernel(toks_hbm, c3_hbm, clast_hbm, emb_hbm, dec_hbm,
                  m_hbm, q_hbm, idx_v, rows_v, out_v,
                  semi0, semi1, semg0, semg1, semo0, semo1):
        wid = lax.axis_index("s") * NC + lax.axis_index("c")
        base_pair = wid * ppw
        semi = (semi0, semi1)
        semg = (semg0, semg1)
        semo = (semo0, semo1)

        qbase = wid * qpw
        pltpu.sync_copy(dec_hbm.at[pl.ds(qbase, qpw)],
                        idx_v.at[0, pl.ds(0, qpw)])
        pltpu.async_copy(emb_hbm.at[idx_v.at[0, pl.ds(0, qpw)]],
                         rows_v.at[0, pl.ds(0, qpw)], semg0).wait()
        pltpu.sync_copy(rows_v.at[0, pl.ds(0, qpw)],
                        q_hbm.at[pl.ds(qbase, qpw)])

        def table(t):
            return c3_hbm.at[t] if t < 3 else clast_hbm

        def idx_src(c):
            return toks_hbm.at[pl.ds((base_pair + c * CK) * T, ckt)]

        def fire_idx(c, ib):
            pltpu.async_copy(idx_src(c), idx_v.at[ib], semi[ib])

        def wait_idx(c, ib):
            pltpu.make_async_copy(idx_src(c), idx_v.at[ib], semi[ib]).wait()

        def fire(c, t, p, ib):
            for j in range(ndma):
                pltpu.async_copy(
                    table(t).at[idx_v.at[ib, pl.ds(j * IDX_DMA, IDX_DMA)]],
                    rows_v.at[p, pl.ds(j * IDX_DMA, IDX_DMA)], semg[p])

        def compute(c, t, p, guard_first):
            for j in range(ndma):
                pltpu.make_async_copy(
                    table(t).at[idx_v.at[0, pl.ds(j * IDX_DMA, IDX_DMA)]],
                    rows_v.at[p, pl.ds(j * IDX_DMA, IDX_DMA)], semg[p]).wait()
            pair0 = base_pair + c * CK
            dst = m_hbm.at[t, pl.ds(pair0, CK)]

            def drain_out():
                pltpu.make_async_copy(out_v.at[p], dst, semo[p]).wait()

            if guard_first is None:
                drain_out()
            else:
                pl.when(guard_first > 0)(drain_out)

            def pair_body(k, _):
                r = k * T
                for dd in range(D // 32):
                    sl = pl.ds(dd * 32, 32)
                    acc_a = None
                    acc_b = None
                    for j in range(0, T, 2):
                        s = rows_v[p, r + j, sl] + rows_v[p, r + j + 1, sl]
                        a, b = plsc.unpack(s, format=plsc.PackFormat.INTERLEAVED)
                        acc_a = a if acc_a is None else acc_a + a
                        acc_b = b if acc_b is None else acc_b + b
                    out_v[p, k, sl] = plsc.pack(
                        acc_a, acc_b, format=plsc.PackFormat.INTERLEAVED)
                return 0

            lax.fori_loop(0, CK, pair_body, 0)
            pltpu.async_copy(out_v.at[p], dst, semo[p])

        fire_idx(0, 0)
        wait_idx(0, 0)
        fire(0, 0, 0, 0)

        def iter_body(c2, _):
            ca = 2 * c2
            cb = ca + 1
            fire_idx(cb, 1)
            fire(ca, 1, 1, 0)
            compute(ca, 0, 0, c2)
            fire(ca, 2, 0, 0)
            compute(ca, 1, 1, c2)
            fire(ca, 3, 1, 0)
            compute(ca, 2, 0, None)
            wait_idx(cb, 1)
            fire(cb, 0, 0, 1)
            compute(ca, 3, 1, None)
            fire(cb, 1, 1, 1)
            compute(cb, 0, 0, None)
            fire(cb, 2, 0, 1)
            compute(cb, 1, 1, None)
            fire(cb, 3, 1, 1)
            compute(cb, 2, 0, None)

            def prefetch_next():
                cn = ca + 2
                fire_idx(cn, 0)
                wait_idx(cn, 0)
                fire(cn, 0, 0, 0)

            pl.when(c2 + 1 < nchunk // 2)(prefetch_next)
            compute(cb, 3, 1, None)
            return 0

        lax.fori_loop(0, nchunk // 2, iter_body, 0)

        for p in range(2):
            pltpu.make_async_copy(out_v.at[p],
                                  m_hbm.at[3, pl.ds(base_pair, CK)],
                                  semo[p]).wait()

    return sc_kernel(toks, c3, clast, emb, dec)


def _tc_attention(m4, embed_q, h0, wi, wh, bi, bh, fw1, fw2, fb, *, B, L, D):
    BB = 32
    grid = (B // BB,)

    def body(m_ref, q_ref, h0_ref, wi_ref, wh_ref, bi_ref, bh_ref,
             fw1_ref, fw2_ref, fb_ref, pptr_ref, pvoc_ref, h1_ref):
        f32 = jnp.float32
        q = q_ref[...].astype(f32)
        h0b = h0_ref[...]
        gi = lax.dot_general(q, wi_ref[...], (((1,), (1,)), ((), ())),
                             preferred_element_type=f32) + bi_ref[...]
        gh = lax.dot_general(h0b, wh_ref[...], (((1,), (1,)), ((), ())),
                             preferred_element_type=f32) + bh_ref[...]
        ir, iz, inn = gi[:, :D], gi[:, D:2 * D], gi[:, 2 * D:]
        hr, hz, hn = gh[:, :D], gh[:, D:2 * D], gh[:, 2 * D:]
        r = jax.nn.sigmoid(ir + hr)
        z = jax.nn.sigmoid(iz + hz)
        n = jnp.tanh(inn + r * hn)
        h1 = (1.0 - z) * n + z * h0b

        u = h1
        scores = None
        for hop in range(3):
            mA = m_ref[hop].astype(f32)
            scores = jnp.sum(mA * u[:, None, :], axis=-1)
            mx = jnp.max(scores, axis=1, keepdims=True)
            e = jnp.exp(scores - mx)
            prob = e / jnp.sum(e, axis=1, keepdims=True)
            mC = m_ref[hop + 1].astype(f32)
            o = jnp.sum(prob[:, :, None] * mC, axis=1)
            if hop == 0:
                logits = (lax.dot_general(h1, fw1_ref[...],
                                          (((1,), (1,)), ((), ())),
                                          preferred_element_type=f32)
                          + lax.dot_general(o, fw2_ref[...],
                                            (((1,), (1,)), ((), ())),
                                            preferred_element_type=f32)
                          + fb_ref[...])
                lmx = jnp.max(logits, axis=1, keepdims=True)
                le = jnp.exp(logits - lmx)
                pvoc_ref[...] = le / jnp.sum(le, axis=1, keepdims=True)
            u = u + o
        pptr_ref[...] = scores
        h1_ref[...] = h1

    full = lambda shape: pl.BlockSpec(shape, lambda i: (0,) * len(shape))
    return pl.pallas_call(
        body,
        grid=grid,
        in_specs=[
            pl.BlockSpec((4, BB, L, D), lambda i: (0, i, 0, 0)),
            pl.BlockSpec((BB, D), lambda i: (i, 0)),
            pl.BlockSpec((BB, D), lambda i: (i, 0)),
            full((3 * D, D)),
            full((3 * D, D)),
            full((1, 3 * D)),
            full((1, 3 * D)),
            full((D, D)),
            full((D, D)),
            full((1, D)),
        ],
        out_specs=[
            pl.BlockSpec((BB, L), lambda i: (i, 0)),
            pl.BlockSpec((BB, D), lambda i: (i, 0)),
            pl.BlockSpec((BB, D), lambda i: (i, 0)),
        ],
        out_shape=[
            jax.ShapeDtypeStruct((B, L), jnp.float32),
            jax.ShapeDtypeStruct((B, D), jnp.float32),
            jax.ShapeDtypeStruct((B, D), jnp.float32),
        ],
    )(m4, embed_q, h0, wi, wh, bi, bh, fw1, fw2, fb)


def kernel(decoder_input, story, hidden_states, embedding, C_tables, C_last,
           gru_Wi, gru_Wh, gru_bi, gru_bh, fc_W, fc_b):
    V, D = embedding.shape
    B, L, T = story.shape
    BL = B * L

    flat = story.reshape(B * L * T).astype(jnp.int32) * 2
    dec = decoder_input.astype(jnp.int32) * 2
    bf = jnp.bfloat16

    def pad2(A):
        widths = [(0, 0)] * (A.ndim - 1) + [(0, D)]
        P = jnp.pad(A.astype(bf), widths)
        return P.reshape(P.shape[:-2] + (2 * P.shape[-2], D))

    m_flat, embed_q = _sc_gather_pool(
        flat, pad2(C_tables), pad2(C_last), pad2(embedding),
        dec, V=V, D=D, T=T, BL=BL, B=B)
    m4 = m_flat.reshape(4, B, L, D)

    h0 = hidden_states[0]
    fw1 = fc_W[:, :D]
    fw2 = fc_W[:, D:]
    p_ptr, p_vocab, h1 = _tc_attention(
        m4, embed_q, h0, gru_Wi, gru_Wh,
        gru_bi.reshape(1, 3 * D), gru_bh.reshape(1, 3 * D),
        fw1, fw2, fc_b.reshape(1, D), B=B, L=L, D=D)

    return (p_ptr, p_vocab, h1[None, :, :])


# speedup vs baseline: 1.1902x; 1.1902x over previous
import functools

import jax
import jax.numpy as jnp
from jax import lax
from jax.experimental import pallas as pl
from jax.experimental.pallas import tpu as pltpu
from jax.experimental.pallas import tpu_sc as plsc

NC = 2
NS = 16
NW = NC * NS

CK = 100
IDX_DMA = 128
LPAD = 208


def _sc_gather_pool(toks, c3, clast, emb, dec, *, V, D, T, L, B):
    bpw = B // NW
    ckt = CK * T
    qpw = B // NW
    sub = []
    o = 0
    while o < ckt:
        sub.append((o, min(IDX_DMA, ckt - o)))
        o += IDX_DMA

    mesh = plsc.VectorSubcoreMesh(core_axis_name="c", subcore_axis_name="s",
                                  num_cores=NC, num_subcores=NS)

    @functools.partial(
        pl.kernel,
        out_type=[jax.ShapeDtypeStruct((2, B, LPAD, 2 * D), jnp.bfloat16),
                  jax.ShapeDtypeStruct((B, D), jnp.bfloat16)],
        mesh=mesh,
        compiler_params=pltpu.CompilerParams(use_tc_tiling_on_sc=False,
                                             needs_layout_passes=False),
        scratch_types=[
            pltpu.VMEM((2, ckt), jnp.int32),
            pltpu.VMEM((2, ckt, D), jnp.bfloat16),
            pltpu.VMEM((2, CK, D), jnp.bfloat16),
            pltpu.SemaphoreType.DMA,
            pltpu.SemaphoreType.DMA,
            pltpu.SemaphoreType.DMA,
            pltpu.SemaphoreType.DMA,
            pltpu.SemaphoreType.DMA,
            pltpu.SemaphoreType.DMA,
        ],
    )
    def sc_kernel(toks_hbm, c3_hbm, clast_hbm, emb_hbm, dec_hbm,
                  m_hbm, q_hbm, idx_v, rows_v, out_v,
                  semi0, semi1, semg0, semg1, semo0, semo1):
        wid = lax.axis_index("s") * NC + lax.axis_index("c")
        base_b = wid * bpw
        semi = (semi0, semi1)
        semg = (semg0, semg1)
        semo = (semo0, semo1)

        qbase = wid * qpw
        pltpu.sync_copy(dec_hbm.at[pl.ds(qbase, qpw)],
                        idx_v.at[0, pl.ds(0, qpw)])
        pltpu.async_copy(emb_hbm.at[idx_v.at[0, pl.ds(0, qpw)]],
                         rows_v.at[0, pl.ds(0, qpw)], semg0).wait()
        pltpu.sync_copy(rows_v.at[0, pl.ds(0, qpw)],
                        q_hbm.at[pl.ds(qbase, qpw)])

        def table(t):
            return c3_hbm.at[t] if t < 3 else clast_hbm

        def idx_src(bl, half):
            return toks_hbm.at[pl.ds(((base_b + bl) * L + half * CK) * T, ckt)]

        def fire_idx(bl, half, ib):
            pltpu.async_copy(idx_src(bl, half), idx_v.at[ib], semi[ib])

        def wait_idx(bl, half, ib):
            pltpu.make_async_copy(idx_src(bl, half), idx_v.at[ib],
                                  semi[ib]).wait()

        def fire(t, p, ib):
            for (off, ln) in sub:
                pltpu.async_copy(
                    table(t).at[idx_v.at[ib, pl.ds(off, ln)]],
                    rows_v.at[p, pl.ds(off, ln)], semg[p])

        def compute(bl, half, t, p, guard_first):
            for (off, ln) in sub:
                pltpu.make_async_copy(
                    table(t).at[idx_v.at[0, pl.ds(off, ln)]],
                    rows_v.at[p, pl.ds(off, ln)], semg[p]).wait()
            dst = m_hbm.at[t // 2, base_b + bl, pl.ds(half * CK, CK),
                           pl.ds((t % 2) * D, D)]

            def drain_out():
                pltpu.make_async_copy(out_v.at[p], dst, semo[p]).wait()

            if guard_first is None:
                drain_out()
            else:
                pl.when(guard_first > 0)(drain_out)

            def pair_body(k, _):
                r = k * T
                for dd in range(D // 32):
                    sl = pl.ds(dd * 32, 32)
                    acc_a = None
                    acc_b = None
                    for j in range(0, T, 2):
                        s = rows_v[p, r + j, sl] + rows_v[p, r + j + 1, sl]
                        a, b = plsc.unpack(s, format=plsc.PackFormat.INTERLEAVED)
                        acc_a = a if acc_a is None else acc_a + a
                        acc_b = b if acc_b is None else acc_b + b
                    out_v[p, k, sl] = plsc.pack(
                        acc_a, acc_b, format=plsc.PackFormat.INTERLEAVED)
                return 0

            lax.fori_loop(0, CK, pair_body, 0)
            pltpu.async_copy(out_v.at[p], dst, semo[p])

        fire_idx(0, 0, 0)
        wait_idx(0, 0, 0)
        fire(0, 0, 0)

        def iter_body(bl, _):
            fire_idx(bl, 1, 1)
            fire(1, 1, 0)
            compute(bl, 0, 0, 0, bl)
            fire(2, 0, 0)
            compute(bl, 0, 1, 1, bl)
            fire(3, 1, 0)
            compute(bl, 0, 2, 0, None)
            wait_idx(bl, 1, 1)
            fire(0, 0, 1)
            compute(bl, 0, 3, 1, None)
            fire(1, 1, 1)
            compute(bl, 1, 0, 0, None)
            fire(2, 0, 1)
            compute(bl, 1, 1, 1, None)
            fire(3, 1, 1)
            compute(bl, 1, 2, 0, None)

            def prefetch_next():
                fire_idx(bl + 1, 0, 0)
                wait_idx(bl + 1, 0, 0)
                fire(0, 0, 0)

            pl.when(bl + 1 < bpw)(prefetch_next)
            compute(bl, 1, 3, 1, None)
            return 0

        lax.fori_loop(0, bpw, iter_body, 0)

        for p in range(2):
            pltpu.make_async_copy(
                out_v.at[p],
                m_hbm.at[0, base_b, pl.ds(0, CK), pl.ds(0, D)],
                semo[p]).wait()

    return sc_kernel(toks, c3, clast, emb, dec)


def _tc_attention(m2, embed_q, h0, wi, wh, bi, bh, fw1, fw2, fb, *, B, L, D):
    BB = 32
    grid = (B // BB,)

    def body(m_ref, q_ref, h0_ref, wi_ref, wh_ref, bi_ref, bh_ref,
             fw1_ref, fw2_ref, fb_ref, pptr_ref, pvoc_ref, h1_ref):
        f32 = jnp.float32
        q = q_ref[...].astype(f32)
        h0b = h0_ref[...]
        gi = lax.dot_general(q, wi_ref[...], (((1,), (1,)), ((), ())),
                             preferred_element_type=f32) + bi_ref[...]
        gh = lax.dot_general(h0b, wh_ref[...], (((1,), (1,)), ((), ())),
                             preferred_element_type=f32) + bh_ref[...]
        ir, iz, inn = gi[:, :D], gi[:, D:2 * D], gi[:, 2 * D:]
        hr, hz, hn = gh[:, :D], gh[:, D:2 * D], gh[:, 2 * D:]
        r = jax.nn.sigmoid(ir + hr)
        z = jax.nn.sigmoid(iz + hz)
        n = jnp.tanh(inn + r * hn)
        h1 = (1.0 - z) * n + z * h0b

        M0 = m_ref[0, :, :L, :].astype(f32)
        M1 = m_ref[1, :, :L, :].astype(f32)
        zpad = jnp.zeros_like(h1)

        u = h1
        scores = None
        plan = [(M0, 0, M0, 1), (M0, 1, M1, 0), (M1, 0, M1, 1)]
        for hop, (MA, uhalf, MC, ohalf) in enumerate(plan):
            ue = (jnp.concatenate([u, zpad], axis=-1) if uhalf == 0
                  else jnp.concatenate([zpad, u], axis=-1))
            scores = jnp.sum(MA * ue[:, None, :], axis=-1)
            mx = jnp.max(scores, axis=1, keepdims=True)
            e = jnp.exp(scores - mx)
            prob = e / jnp.sum(e, axis=1, keepdims=True)
            w = jnp.sum(prob[:, :, None] * MC, axis=1)
            o = w[:, D:] if ohalf else w[:, :D]
            if hop == 0:
                logits = (lax.dot_general(h1, fw1_ref[...],
                                          (((1,), (1,)), ((), ())),
                                          preferred_element_type=f32)
                          + lax.dot_general(o, fw2_ref[...],
                                            (((1,), (1,)), ((), ())),
                                            preferred_element_type=f32)
                          + fb_ref[...])
                lmx = jnp.max(logits, axis=1, keepdims=True)
                le = jnp.exp(logits - lmx)
                pvoc_ref[...] = le / jnp.sum(le, axis=1, keepdims=True)
            u = u + o
        pptr_ref[...] = scores
        h1_ref[...] = h1

    full = lambda shape: pl.BlockSpec(shape, lambda i: (0,) * len(shape))
    return pl.pallas_call(
        body,
        grid=grid,
        in_specs=[
            pl.BlockSpec((2, BB, LPAD, 2 * D), lambda i: (0, i, 0, 0)),
            pl.BlockSpec((BB, D), lambda i: (i, 0)),
            pl.BlockSpec((BB, D), lambda i: (i, 0)),
            full((3 * D, D)),
            full((3 * D, D)),
            full((1, 3 * D)),
            full((1, 3 * D)),
            full((D, D)),
            full((D, D)),
            full((1, D)),
        ],
        out_specs=[
            pl.BlockSpec((BB, L), lambda i: (i, 0)),
            pl.BlockSpec((BB, D), lambda i: (i, 0)),
            pl.BlockSpec((BB, D), lambda i: (i, 0)),
        ],
        out_shape=[
            jax.ShapeDtypeStruct((B, L), jnp.float32),
            jax.ShapeDtypeStruct((B, D), jnp.float32),
            jax.ShapeDtypeStruct((B, D), jnp.float32),
        ],
    )(m2, embed_q, h0, wi, wh, bi, bh, fw1, fw2, fb)


def kernel(decoder_input, story, hidden_states, embedding, C_tables, C_last,
           gru_Wi, gru_Wh, gru_bi, gru_bh, fc_W, fc_b):
    V, D = embedding.shape
    B, L, T = story.shape

    flat = story.reshape(B * L * T).astype(jnp.int32)
    dec = decoder_input.astype(jnp.int32)
    bf = jnp.bfloat16

    m2, embed_q = _sc_gather_pool(
        flat, C_tables.astype(bf), C_last.astype(bf), embedding.astype(bf),
        dec, V=V, D=D, T=T, L=L, B=B)

    h0 = hidden_states[0]
    fw1 = fc_W[:, :D]
    fw2 = fc_W[:, D:]
    p_ptr, p_vocab, h1 = _tc_attention(
        m2, embed_q, h0, gru_Wi, gru_Wh,
        gru_bi.reshape(1, 3 * D), gru_bh.reshape(1, 3 * D),
        fw1, fw2, fc_b.reshape(1, D), B=B, L=L, D=D)

    return (p_ptr, p_vocab, h1[None, :, :])


# speedup vs baseline: 1.5782x; 1.3260x over previous
import functools

import jax
import jax.numpy as jnp
from jax import lax
from jax.experimental import pallas as pl
from jax.experimental.pallas import tpu as pltpu
from jax.experimental.pallas import tpu_sc as plsc

NC = 2
NS = 16
NW = NC * NS

CK = 100
IDX_DMA = 128
LPAD = 200


def _sc_gather_pool(toks, c3, clast, emb, dec, *, V, D, T, L, B):
    bpw = B // NW
    ckt = CK * T
    qpw = B // NW
    sub = []
    o = 0
    while o < ckt:
        sub.append((o, min(IDX_DMA, ckt - o)))
        o += IDX_DMA

    mesh = plsc.VectorSubcoreMesh(core_axis_name="c", subcore_axis_name="s",
                                  num_cores=NC, num_subcores=NS)

    @functools.partial(
        pl.kernel,
        out_type=[jax.ShapeDtypeStruct((2, B, LPAD, 2 * D), jnp.float32),
                  jax.ShapeDtypeStruct((B, D), jnp.bfloat16)],
        mesh=mesh,
        compiler_params=pltpu.CompilerParams(use_tc_tiling_on_sc=False,
                                             needs_layout_passes=False),
        scratch_types=[
            pltpu.VMEM((2, ckt), jnp.int32),
            pltpu.VMEM((2, ckt, D), jnp.bfloat16),
            pltpu.VMEM((2, CK, D), jnp.float32),
            pltpu.SemaphoreType.DMA,
            pltpu.SemaphoreType.DMA,
            pltpu.SemaphoreType.DMA,
            pltpu.SemaphoreType.DMA,
            pltpu.SemaphoreType.DMA,
            pltpu.SemaphoreType.DMA,
        ],
    )
    def sc_kernel(toks_hbm, c3_hbm, clast_hbm, emb_hbm, dec_hbm,
                  m_hbm, q_hbm, idx_v, rows_v, out_v,
                  semi0, semi1, semg0, semg1, semo0, semo1):
        wid = lax.axis_index("s") * NC + lax.axis_index("c")
        base_b = wid * bpw
        semi = (semi0, semi1)
        semg = (semg0, semg1)
        semo = (semo0, semo1)

        qbase = wid * qpw
        pltpu.sync_copy(dec_hbm.at[pl.ds(qbase, qpw)],
                        idx_v.at[0, pl.ds(0, qpw)])
        pltpu.async_copy(emb_hbm.at[idx_v.at[0, pl.ds(0, qpw)]],
                         rows_v.at[0, pl.ds(0, qpw)], semg0).wait()
        pltpu.sync_copy(rows_v.at[0, pl.ds(0, qpw)],
                        q_hbm.at[pl.ds(qbase, qpw)])

        def table(t):
            return c3_hbm.at[t] if t < 3 else clast_hbm

        def idx_src(bl, half):
            return toks_hbm.at[pl.ds(((base_b + bl) * L + half * CK) * T, ckt)]

        def fire_idx(bl, half, ib):
            pltpu.async_copy(idx_src(bl, half), idx_v.at[ib], semi[ib])

        def wait_idx(bl, half, ib):
            pltpu.make_async_copy(idx_src(bl, half), idx_v.at[ib],
                                  semi[ib]).wait()

        def fire(t, p, ib):
            for (off, ln) in sub:
                pltpu.async_copy(
                    table(t).at[idx_v.at[ib, pl.ds(off, ln)]],
                    rows_v.at[p, pl.ds(off, ln)], semg[p])

        def compute(bl, half, t, p, guard_first):
            for (off, ln) in sub:
                pltpu.make_async_copy(
                    table(t).at[idx_v.at[0, pl.ds(off, ln)]],
                    rows_v.at[p, pl.ds(off, ln)], semg[p]).wait()
            dst = m_hbm.at[t // 2, base_b + bl, pl.ds(half * CK, CK),
                           pl.ds((t % 2) * D, D)]

            def drain_out():
                pltpu.make_async_copy(out_v.at[p], dst, semo[p]).wait()

            if guard_first is None:
                drain_out()
            else:
                pl.when(guard_first > 0)(drain_out)

            def pair_body(k, _):
                r = k * T
                for dd in range(D // 32):
                    sl = pl.ds(dd * 32, 32)
                    acc_a = None
                    acc_b = None
                    for j in range(0, T, 2):
                        s = rows_v[p, r + j, sl] + rows_v[p, r + j + 1, sl]
                        a, b = plsc.unpack(s, format=plsc.PackFormat.INTERLEAVED)
                        acc_a = a if acc_a is None else acc_a + a
                        acc_b = b if acc_b is None else acc_b + b
                    out_v[p, k, pl.ds(dd * 32, 16)] = acc_a
                    out_v[p, k, pl.ds(dd * 32 + 16, 16)] = acc_b
                return 0

            lax.fori_loop(0, CK, pair_body, 0)
            pltpu.async_copy(out_v.at[p], dst, semo[p])

        fire_idx(0, 0, 0)
        wait_idx(0, 0, 0)
        fire(0, 0, 0)

        def iter_body(bl, _):
            fire_idx(bl, 1, 1)
            fire(1, 1, 0)
            compute(bl, 0, 0, 0, bl)
            fire(2, 0, 0)
            compute(bl, 0, 1, 1, bl)
            fire(3, 1, 0)
            compute(bl, 0, 2, 0, None)
            wait_idx(bl, 1, 1)
            fire(0, 0, 1)
            compute(bl, 0, 3, 1, None)
            fire(1, 1, 1)
            compute(bl, 1, 0, 0, None)
            fire(2, 0, 1)
            compute(bl, 1, 1, 1, None)
            fire(3, 1, 1)
            compute(bl, 1, 2, 0, None)

            def prefetch_next():
                fire_idx(bl + 1, 0, 0)
                wait_idx(bl + 1, 0, 0)
                fire(0, 0, 0)

            pl.when(bl + 1 < bpw)(prefetch_next)
            compute(bl, 1, 3, 1, None)
            return 0

        lax.fori_loop(0, bpw, iter_body, 0)

        for p in range(2):
            pltpu.make_async_copy(
                out_v.at[p],
                m_hbm.at[0, base_b, pl.ds(0, CK), pl.ds(0, D)],
                semo[p]).wait()

    return sc_kernel(toks, c3, clast, emb, dec)


def _tc_attention(m2, embed_q, h0, wi, wh, bi, bh, fw1, fw2p, fb, pmat,
                  *, B, L, D):
    BB = 32
    grid = (B // BB,)

    def body(m_ref, q_ref, h0_ref, wi_ref, wh_ref, bi_ref, bh_ref,
             fw1_ref, fw2_ref, fb_ref, pmat_ref, pptr_ref, pvoc_ref,
             h1_ref):
        f32 = jnp.float32
        q = q_ref[...].astype(f32)
        h0b = h0_ref[...]
        gi = lax.dot_general(q, wi_ref[...], (((1,), (1,)), ((), ())),
                             preferred_element_type=f32) + bi_ref[...]
        gh = lax.dot_general(h0b, wh_ref[...], (((1,), (1,)), ((), ())),
                             preferred_element_type=f32) + bh_ref[...]
        ir, iz, inn = gi[:, :D], gi[:, D:2 * D], gi[:, 2 * D:]
        hr, hz, hn = gh[:, :D], gh[:, D:2 * D], gh[:, 2 * D:]
        r = jax.nn.sigmoid(ir + hr)
        z = jax.nn.sigmoid(iz + hz)
        n = jnp.tanh(inn + r * hn)
        h1 = (1.0 - z) * n + z * h0b

        M0 = m_ref[0]
        M1 = m_ref[1]
        zpad = jnp.zeros_like(h1)

        u = lax.dot_general(h1, pmat_ref[...], (((1,), (1,)), ((), ())),
                            preferred_element_type=f32)
        scores = None
        plan = [(M0, 0, M0, 1), (M0, 1, M1, 0), (M1, 0, M1, 1)]
        for hop, (MA, uhalf, MC, ohalf) in enumerate(plan):
            ue = (jnp.concatenate([u, zpad], axis=-1) if uhalf == 0
                  else jnp.concatenate([zpad, u], axis=-1))
            scores = jnp.sum(MA * ue[:, None, :], axis=-1)
            mx = jnp.max(scores, axis=1, keepdims=True)
            e = jnp.exp(scores - mx)
            prob = e / jnp.sum(e, axis=1, keepdims=True)
            w = jnp.sum(prob[:, :, None] * MC, axis=1)
            o = w[:, D:] if ohalf else w[:, :D]
            if hop == 0:
                logits = (lax.dot_general(h1, fw1_ref[...],
                                          (((1,), (1,)), ((), ())),
                                          preferred_element_type=f32)
                          + lax.dot_general(o, fw2_ref[...],
                                            (((1,), (1,)), ((), ())),
                                            preferred_element_type=f32)
                          + fb_ref[...])
                lmx = jnp.max(logits, axis=1, keepdims=True)
                le = jnp.exp(logits - lmx)
                pvoc_ref[...] = le / jnp.sum(le, axis=1, keepdims=True)
            u = u + o
        pptr_ref[...] = scores
        h1_ref[...] = h1

    full = lambda shape: pl.BlockSpec(shape, lambda i: (0,) * len(shape))
    return pl.pallas_call(
        body,
        grid=grid,
        in_specs=[
            pl.BlockSpec((2, BB, LPAD, 2 * D), lambda i: (0, i, 0, 0)),
            pl.BlockSpec((BB, D), lambda i: (i, 0)),
            pl.BlockSpec((BB, D), lambda i: (i, 0)),
            full((3 * D, D)),
            full((3 * D, D)),
            full((1, 3 * D)),
            full((1, 3 * D)),
            full((D, D)),
            full((D, D)),
            full((1, D)),
            full((D, D)),
        ],
        out_specs=[
            pl.BlockSpec((BB, L), lambda i: (i, 0)),
            pl.BlockSpec((BB, D), lambda i: (i, 0)),
            pl.BlockSpec((BB, D), lambda i: (i, 0)),
        ],
        out_shape=[
            jax.ShapeDtypeStruct((B, L), jnp.float32),
            jax.ShapeDtypeStruct((B, D), jnp.float32),
            jax.ShapeDtypeStruct((B, D), jnp.float32),
        ],
    )(m2, embed_q, h0, wi, wh, bi, bh, fw1, fw2p, fb, pmat)


def kernel(decoder_input, story, hidden_states, embedding, C_tables, C_last,
           gru_Wi, gru_Wh, gru_bi, gru_bh, fc_W, fc_b):
    V, D = embedding.shape
    B, L, T = story.shape

    flat = story.reshape(B * L * T).astype(jnp.int32)
    dec = decoder_input.astype(jnp.int32)
    bf = jnp.bfloat16

    m2, embed_q = _sc_gather_pool(
        flat, C_tables.astype(bf), C_last.astype(bf), embedding.astype(bf),
        dec, V=V, D=D, T=T, L=L, B=B)

    h0 = hidden_states[0]
    fw1 = fc_W[:, :D]
    perm = []
    for blk in range(0, D, 32):
        perm += [blk + 2 * i for i in range(16)]
        perm += [blk + 2 * i + 1 for i in range(16)]
    perm = jnp.array(perm, dtype=jnp.int32)
    pmat = jnp.eye(D, dtype=jnp.float32)[perm]
    fw2p = fc_W[:, D:][:, perm]
    p_ptr, p_vocab, h1 = _tc_attention(
        m2, embed_q, h0, gru_Wi, gru_Wh,
        gru_bi.reshape(1, 3 * D), gru_bh.reshape(1, 3 * D),
        fw1, fw2p, fc_b.reshape(1, D), pmat, B=B, L=L, D=D)

    return (p_ptr, p_vocab, h1[None, :, :])
